# baseline (speedup 1.0000x reference)
import functools
import jax
import jax.numpy as jnp
from jax.experimental import pallas as pl
from jax.experimental.pallas import tpu as pltpu

N = 10000
E = 320000
D = 128
NUM_LAYERS = 2
PROT_LEN = 2000
EPS = 1e-5
ETYPES = ['knn', 'rsphere', 'seq']
ROWS = 1000


def _g_body(h_ref, nsrc_ref, w_ref, g_ref):
    hb = h_ref[:]
    for et in range(3):
        hs = hb * nsrc_ref[:, et][:, None]
        g_ref[et] = jnp.dot(hs, w_ref[et], preferred_element_type=jnp.float32)


def _g_call(h, nsrc, W3):
    grid = (N // ROWS,)
    return pl.pallas_call(
        _g_body,
        grid=grid,
        in_specs=[
            pl.BlockSpec((ROWS, D), lambda i: (i, 0)),
            pl.BlockSpec((ROWS, 3), lambda i: (i, 0)),
            pl.BlockSpec((3, D, D), lambda i: (0, 0, 0)),
        ],
        out_specs=pl.BlockSpec((3, ROWS, D), lambda i: (0, i, 0)),
        out_shape=jax.ShapeDtypeStruct((3, N, D), jnp.float32),
    )(h, nsrc, W3)


def _epi1_body(aggs_ref, ndst_ref, pre_ref, stats_ref):
    i = pl.program_id(0)
    pre = jnp.zeros((ROWS, D), jnp.float32)
    for et in range(3):
        a = aggs_ref[et, 0] + aggs_ref[et, 1]
        pre = pre + a * ndst_ref[:, et][:, None]
    pre_ref[:] = pre

    @pl.when(i == 0)
    def _():
        stats_ref[:] = jnp.zeros_like(stats_ref)

    stats_ref[0] += jnp.sum(pre, axis=0)
    stats_ref[1] += jnp.sum(pre * pre, axis=0)


def _epi1_call(aggs, ndst):
    grid = (N // ROWS,)
    return pl.pallas_call(
        _epi1_body,
        grid=grid,
        in_specs=[
            pl.BlockSpec((3, 2, ROWS, D), lambda i: (0, 0, i, 0)),
            pl.BlockSpec((ROWS, 3), lambda i: (i, 0)),
        ],
        out_specs=[
            pl.BlockSpec((ROWS, D), lambda i: (i, 0)),
            pl.BlockSpec((2, D), lambda i: (0, 0)),
        ],
        out_shape=[
            jax.ShapeDtypeStruct((N, D), jnp.float32),
            jax.ShapeDtypeStruct((2, D), jnp.float32),
        ],
    )(aggs, ndst)


def _epi2_body(pre_ref, stats_ref, gb_ref, fcw_ref, fcb_ref, out_ref):
    mean = stats_ref[0] / N
    var = stats_ref[1] / N - mean * mean
    scale = jax.lax.rsqrt(var + EPS) * gb_ref[0]
    shift = gb_ref[1] - mean * scale
    y = jnp.maximum(pre_ref[:] * scale[None, :] + shift[None, :], 0.0)
    out_ref[:] = jnp.dot(y, fcw_ref[:], preferred_element_type=jnp.float32) \
        + fcb_ref[0][None, :]


def _epi2_call(pre, stats, gamma, beta, fcw, fcb):
    grid = (N // ROWS,)
    gb = jnp.stack([gamma, beta])
    return pl.pallas_call(
        _epi2_body,
        grid=grid,
        in_specs=[
            pl.BlockSpec((ROWS, D), lambda i: (i, 0)),
            pl.BlockSpec((2, D), lambda i: (0, 0)),
            pl.BlockSpec((2, D), lambda i: (0, 0)),
            pl.BlockSpec((D, D), lambda i: (0, 0)),
            pl.BlockSpec((1, D), lambda i: (0, 0)),
        ],
        out_specs=pl.BlockSpec((ROWS, D), lambda i: (i, 0)),
        out_shape=jax.ShapeDtypeStruct((N, D), jnp.float32),
    )(pre, stats, gb, fcw, fcb.reshape(1, D))


def _degrees(edges3):
    degs = []
    for which in range(2):
        per = []
        for et in range(3):
            d = jnp.zeros((N,), jnp.float32).at[edges3[et, which]].add(1.0)
            per.append(d)
        degs.append(jnp.stack(per))
    return jnp.stack(degs)


def _scatter_aggs(g3, edges3):
    outs = []
    for et in range(3):
        src = edges3[et, 0]
        dst = edges3[et, 1]
        acc = jnp.zeros((N, D), jnp.float32).at[dst].add(g3[et][src])
        outs.append(jnp.stack([acc, jnp.zeros((N, D), jnp.float32)]))
    return jnp.stack(outs)


@jax.jit
def _forward_impl(x, edge_knn, edge_rsphere, edge_seq, params):
    edges3 = jnp.stack([edge_knn, edge_rsphere, edge_seq]).astype(jnp.int32)
    degs = _degrees(edges3)
    norms = jax.lax.rsqrt(jnp.maximum(degs, 1.0))
    nsrc, ndst = norms[0].T, norms[1].T

    h = x
    for i in range(NUM_LAYERS):
        lp = params['layer%d' % i]
        W3 = jnp.stack([lp[et + '_W'] for et in ETYPES])
        g3 = _g_call(h, nsrc, W3)
        aggs = _scatter_aggs(g3, edges3)
        pre, stats = _epi1_call(aggs, ndst)
        h = _epi2_call(pre, stats, lp['bn_gamma'], lp['bn_beta'],
                       lp['fc_W'], lp['fc_b'])
    return h.reshape(-1, PROT_LEN, D)


def kernel(x, edge_knn, edge_rsphere, edge_seq, params):
    return _forward_impl(x, edge_knn, edge_rsphere, edge_seq, params)
